# Initial kernel scaffold; baseline (speedup 1.0000x reference)
import jax
import jax.numpy as jnp
from jax.experimental import pallas as pl


def kernel(x, W1, W2, W3, W4, W5, W6, W7, W8, W9, g1, g2, g3, g4, g5, g6, g7, g8, b1, b2, b3, b4, b5, b6, b7, b8):
    raise NotImplementedError("write your pallas kernel here")


# speedup vs baseline: 4.5495x; 4.5495x over previous
import functools

import jax
import jax.numpy as jnp
from jax import lax
from jax.experimental import pallas as pl
from jax.experimental.pallas import tpu as pltpu
from jax.experimental.pallas import tpu_sc as plsc

KNN_K = 20


def _leaky(y):
    return jnp.where(y >= 0, y, 0.2 * y)


def _knn_body(xr_ref, xt_ref, wu_ref, wv_ref, bv_ref, idx_ref, u_ref, v_ref,
              *, K, N):
    b = pl.program_id(0)
    xr = xr_ref[0]
    xt = xt_ref[0]
    dots = jnp.dot(xr, xt, preferred_element_type=jnp.float32)
    xxr = jnp.sum(xr * xr, axis=1, keepdims=True)
    xxc = jnp.sum(xt * xt, axis=0, keepdims=True)
    pd = 2.0 * dots - xxr - xxc
    TN = pd.shape[0]
    col = lax.broadcasted_iota(jnp.int32, (TN, N), 1)
    kcol = lax.broadcasted_iota(jnp.int32, (TN, K), 1)

    def step(t, carry):
        pdc, acc = carry
        m = jnp.max(pdc, axis=1, keepdims=True)
        am = jnp.min(jnp.where(pdc == m, col, N), axis=1, keepdims=True)
        acc = jnp.where(kcol == t, am, acc)
        pdc = jnp.where(col == am, -jnp.inf, pdc)
        return pdc, acc

    _, acc = lax.fori_loop(0, K, step, (pd, jnp.zeros((TN, K), jnp.int32)))
    idx_ref[0] = acc + b * N
    u_ref[0] = jnp.dot(xr, wu_ref[...], preferred_element_type=jnp.float32)
    v_ref[0] = (jnp.dot(xr, wv_ref[...], preferred_element_type=jnp.float32)
                + bv_ref[...])


def _knn(xr, xt, wu, wv, bv, K, TN):
    B, N, C = xr.shape
    return pl.pallas_call(
        functools.partial(_knn_body, K=K, N=N),
        grid=(B, N // TN),
        in_specs=[
            pl.BlockSpec((1, TN, C), lambda b, t: (b, t, 0)),
            pl.BlockSpec((1, C, N), lambda b, t: (b, 0, 0)),
            pl.BlockSpec((C, 128), lambda b, t: (0, 0)),
            pl.BlockSpec((C, 64), lambda b, t: (0, 0)),
            pl.BlockSpec((1, 64), lambda b, t: (0, 0)),
        ],
        out_specs=[
            pl.BlockSpec((1, TN, K), lambda b, t: (b, t, 0)),
            pl.BlockSpec((1, TN, 128), lambda b, t: (b, t, 0)),
            pl.BlockSpec((1, TN, 64), lambda b, t: (b, t, 0)),
        ],
        out_shape=[
            jax.ShapeDtypeStruct((B, N, K), jnp.int32),
            jax.ShapeDtypeStruct((B, N, 128), jnp.float32),
            jax.ShapeDtypeStruct((B, N, 64), jnp.float32),
        ],
    )(xr, xt, wu, wv, bv)


def _make_sc_gather(MK, D, CH):
    info = plsc.get_sparse_core_info()
    NC, NS = info.num_cores, info.num_subcores
    NW = NC * NS
    per_w = MK // NW
    n_ch = per_w // CH
    assert per_w % CH == 0 and MK % NW == 0
    mesh = plsc.VectorSubcoreMesh(core_axis_name="c", subcore_axis_name="s")

    @functools.partial(
        pl.kernel, mesh=mesh,
        out_type=jax.ShapeDtypeStruct((MK, D), jnp.float32),
        scratch_types=[
            pltpu.VMEM((CH,), jnp.int32),
            pltpu.VMEM((CH, D), jnp.float32),
            pltpu.SemaphoreType.DMA,
        ],
    )
    def gk(idx_hbm, table_hbm, out_hbm, idx_v, rows_v, sem):
        wid = lax.axis_index("s") * NC + lax.axis_index("c")
        base0 = wid * per_w

        def body(i, carry):
            base = base0 + i * CH
            pltpu.sync_copy(idx_hbm.at[pl.ds(base, CH)], idx_v)
            pltpu.async_copy(table_hbm.at[idx_v], rows_v, sem).wait()
            pltpu.sync_copy(rows_v, out_hbm.at[pl.ds(base, CH)])
            return carry

        lax.fori_loop(0, n_ch, body, 0)

    return gk


def _edge_body(g_ref, v_ref, w_ref, b_ref, out_ref, *, K):
    v = v_ref[...]
    TNE = v.shape[0]
    g = g_ref[...].reshape(K * TNE, 128)[:, :64]
    h = _leaky(g + jnp.tile(v, (K, 1)))
    y = _leaky(jnp.dot(h, w_ref[...], preferred_element_type=jnp.float32)
               + b_ref[...])
    out_ref[...] = jnp.max(y.reshape(K, TNE, 64), axis=0)


def _edge(g3, v, w, bias, K, TNE):
    M = v.shape[0]
    return pl.pallas_call(
        functools.partial(_edge_body, K=K),
        grid=(M // TNE,),
        in_specs=[
            pl.BlockSpec((K, TNE, 128), lambda t: (0, t, 0)),
            pl.BlockSpec((TNE, 64), lambda t: (t, 0)),
            pl.BlockSpec((64, 64), lambda t: (0, 0)),
            pl.BlockSpec((1, 64), lambda t: (0, 0)),
        ],
        out_specs=pl.BlockSpec((TNE, 64), lambda t: (t, 0)),
        out_shape=jax.ShapeDtypeStruct((M, 64), jnp.float32),
    )(g3, v, w, bias)


def _max3_body(g_ref, v_ref, out_ref, *, K):
    out_ref[...] = _leaky(jnp.max(g_ref[...], axis=0)[:, :64] + v_ref[...])


def _max3(g3, v, K, TNE):
    M = v.shape[0]
    return pl.pallas_call(
        functools.partial(_max3_body, K=K),
        grid=(M // TNE,),
        in_specs=[
            pl.BlockSpec((K, TNE, 128), lambda t: (0, t, 0)),
            pl.BlockSpec((TNE, 64), lambda t: (t, 0)),
        ],
        out_specs=pl.BlockSpec((TNE, 64), lambda t: (t, 0)),
        out_shape=jax.ShapeDtypeStruct((M, 64), jnp.float32),
    )(g3, v)


def _head_body(x1_ref, x2_ref, x3_ref, w6_ref, b6_ref, w7g_ref, w7l_ref,
               b7_ref, w8_ref, b8_ref, w9_ref, out_ref):
    hc = jnp.concatenate([x1_ref[0], x2_ref[0], x3_ref[0]], axis=1)
    h6 = _leaky(jnp.dot(hc, w6_ref[...], preferred_element_type=jnp.float32)
                + b6_ref[...])
    gmax = jnp.max(h6, axis=0, keepdims=True)
    c7 = jnp.dot(gmax, w7g_ref[...], preferred_element_type=jnp.float32)
    h7 = _leaky(jnp.dot(hc, w7l_ref[...], preferred_element_type=jnp.float32)
                + c7 + b7_ref[...])
    h8 = _leaky(jnp.dot(h7, w8_ref[...], preferred_element_type=jnp.float32)
                + b8_ref[...])
    y9 = jnp.dot(h8, w9_ref[...], preferred_element_type=jnp.float32)
    out_ref[0] = 1.0 / (1.0 + jnp.exp(-y9))


def _head(x1, x2, x3, w6, b6, w7g, w7l, b7, w8, b8, w9):
    B, N, _ = x1.shape
    full = lambda shape: pl.BlockSpec(shape, lambda b: tuple(0 for _ in shape))
    return pl.pallas_call(
        _head_body,
        grid=(B,),
        in_specs=[
            pl.BlockSpec((1, N, 64), lambda b: (b, 0, 0)),
            pl.BlockSpec((1, N, 64), lambda b: (b, 0, 0)),
            pl.BlockSpec((1, N, 64), lambda b: (b, 0, 0)),
            full((192, 1024)), full((1, 1024)),
            full((1024, 512)), full((192, 512)), full((1, 512)),
            full((512, 256)), full((1, 256)),
            full((256, 1)),
        ],
        out_specs=pl.BlockSpec((1, N, 1), lambda b: (b, 0, 0)),
        out_shape=jax.ShapeDtypeStruct((B, N, 1), jnp.float32),
    )(x1, x2, x3, w6, b6, w7g, w7l, b7, w8, b8, w9)


def kernel(x, W1, W2, W3, W4, W5, W6, W7, W8, W9,
           g1, g2, g3, g4, g5, g6, g7, g8,
           b1, b2, b3, b4, b5, b6, b7, b8):
    B, C0, N = x.shape
    K = KNN_K
    M = B * N
    sc = 1.0 / jnp.sqrt(jnp.float32(1.0 + 1e-5))

    def split(W, g, C):
        wa, wb = W[:, :C], W[:, C:]
        wu = (wa * (sc * g)[:, None]).T
        wv = ((wb - wa) * (sc * g)[:, None]).T
        return wu, wv

    wu1, wv1 = split(W1, g1, C0)
    wu1 = jnp.pad(wu1, ((0, 8 - C0), (0, 64)))
    wv1 = jnp.pad(wv1, ((0, 8 - C0), (0, 0)))
    wu3, wv3 = split(W3, g3, 64)
    wu5, wv5 = split(W5, g5, 64)
    wu3 = jnp.pad(wu3, ((0, 0), (0, 64)))
    wu5 = jnp.pad(wu5, ((0, 0), (0, 64)))
    w2t = (W2 * (sc * g2)[:, None]).T
    w4t = (W4 * (sc * g4)[:, None]).T
    w6t = (W6 * (sc * g6)[:, None]).T
    w7g = (W7[:, :1024] * (sc * g7)[:, None]).T
    w7l = (W7[:, 1024:] * (sc * g7)[:, None]).T
    w8t = (W8 * (sc * g8)[:, None]).T
    w9t = W9.T
    row = lambda b: b[None, :]

    gather = _make_sc_gather(M * K, 128, 128)
    perm = lambda idx: idx.reshape(M, K).T.reshape(-1)

    xr1 = jnp.pad(jnp.swapaxes(x, 1, 2), ((0, 0), (0, 0), (0, 8 - C0)))
    xt1 = jnp.pad(x, ((0, 0), (0, 8 - C0), (0, 0)))
    idx, u, v = _knn(xr1, xt1, wu1, wv1, row(b1), K, 256)
    g = gather(perm(idx), u.reshape(M, 128)).reshape(K, M, 128)
    x1 = _edge(g, v.reshape(M, 64), w2t, row(b2), K, 256).reshape(B, N, 64)

    idx, u, v = _knn(x1, jnp.swapaxes(x1, 1, 2), wu3, wv3, row(b3), K, 256)
    g = gather(perm(idx), u.reshape(M, 128)).reshape(K, M, 128)
    x2 = _edge(g, v.reshape(M, 64), w4t, row(b4), K, 256).reshape(B, N, 64)

    idx, u, v = _knn(x2, jnp.swapaxes(x2, 1, 2), wu5, wv5, row(b5), K, 256)
    g = gather(perm(idx), u.reshape(M, 128)).reshape(K, M, 128)
    x3 = _max3(g, v.reshape(M, 64), K, 256).reshape(B, N, 64)

    out = _head(x1, x2, x3, w6t, row(b6), w7g, w7l, row(b7),
                w8t, row(b8), w9t)
    return out.reshape(B, N)
